# Initial kernel scaffold; baseline (speedup 1.0000x reference)
import jax
import jax.numpy as jnp
from jax.experimental import pallas as pl


def kernel(h, edge_index, edge_attr, bond_emb, W1, b1, g1, be1, W2, b2, g2, be2, eps_param, add_activation):
    raise NotImplementedError("write your pallas kernel here")


# speedup vs baseline: 7.2509x; 7.2509x over previous
import functools

import jax
import jax.numpy as jnp
from jax import lax
from jax.experimental import pallas as pl
from jax.experimental.pallas import tpu as pltpu
from jax.experimental.pallas import tpu_sc as plsc

N_NODES = 10000
N_EDGES = 320000
EMB = 128
NCORES = 2
NSUB = 16
NW = NCORES * NSUB
EPW = N_EDGES // NW
CHUNK = 80
NCHUNK = EPW // CHUNK
NPAD = 10240
ROWS_PER_TILE = NPAD // NSUB
LANES = 16
SL = EMB // LANES


def _sc_body(h_hbm, src_hbm, cidx_hbm, dst_hbm, ct_hbm, z_hbm, out_hbm,
             srcb, cidxb, dstb, bufh, bufe, ct_sp, aggr_sp, sem1, sem2):
    cid = lax.axis_index("c")
    sid = lax.axis_index("s")
    wid = cid * NSUB + sid

    r0 = sid * ROWS_PER_TILE
    pltpu.sync_copy(z_hbm.at[pl.ds(r0, ROWS_PER_TILE)],
                    aggr_sp.at[pl.ds(r0, ROWS_PER_TILE)])

    @pl.when(sid == 0)
    def _():
        pltpu.sync_copy(ct_hbm, ct_sp)

    plsc.subcore_barrier()

    ebase = wid * EPW

    def step(i, carry):
        base = ebase + i * CHUNK
        pltpu.sync_copy(src_hbm.at[pl.ds(base, CHUNK)], srcb)
        pltpu.sync_copy(cidx_hbm.at[pl.ds(base, CHUNK)], cidxb)
        pltpu.sync_copy(dst_hbm.at[pl.ds(base, CHUNK)], dstb)
        cp1 = pltpu.async_copy(h_hbm.at[srcb], bufh, sem1)
        cp2 = pltpu.async_copy(ct_sp.at[cidxb], bufe, sem2)
        cp1.wait()
        cp2.wait()

        def jstep(j, c2):
            for s in range(SL):
                sl = pl.ds(s * LANES, LANES)
                v = bufh[j, sl] + bufe[j, sl]
                bufh[j, sl] = jnp.maximum(v, 0.0)
            return c2

        lax.fori_loop(0, CHUNK, jstep, 0)
        pltpu.sync_copy(bufh, aggr_sp.at[dstb], add=True)
        return carry

    lax.fori_loop(0, NCHUNK, step, 0)

    plsc.subcore_barrier()
    pltpu.sync_copy(aggr_sp.at[pl.ds(r0, ROWS_PER_TILE)],
                    out_hbm.at[cid, pl.ds(r0, ROWS_PER_TILE)])


def _sc_aggregate(h, src, cidx, dst, ctable, zeros):
    mesh = plsc.VectorSubcoreMesh(core_axis_name="c", subcore_axis_name="s")
    return pl.kernel(
        _sc_body,
        out_type=jax.ShapeDtypeStruct((NCORES, NPAD, EMB), jnp.float32),
        mesh=mesh,
        scratch_types=[
            pltpu.VMEM((CHUNK,), jnp.int32),
            pltpu.VMEM((CHUNK,), jnp.int32),
            pltpu.VMEM((CHUNK,), jnp.int32),
            pltpu.VMEM((CHUNK, EMB), jnp.float32),
            pltpu.VMEM((CHUNK, EMB), jnp.float32),
            pltpu.VMEM_SHARED((512, EMB), jnp.float32),
            pltpu.VMEM_SHARED((NPAD, EMB), jnp.float32),
            pltpu.SemaphoreType.DMA,
            pltpu.SemaphoreType.DMA,
        ],
    )(h, src, cidx, dst, ctable, zeros)


def _ct_body(be_ref, o_ref):
    t0 = be_ref[0]
    t1 = be_ref[1]
    t2 = be_ref[2]
    r0 = jnp.repeat(t0, 64, axis=0)
    r1 = jnp.tile(jnp.repeat(t1, 8, axis=0), (8, 1))
    r2 = jnp.tile(t2, (64, 1))
    o_ref[...] = r0 + r1 + r2


def _build_ctable(bond_emb):
    return pl.pallas_call(
        _ct_body,
        out_shape=jax.ShapeDtypeStruct((512, EMB), jnp.float32),
    )(bond_emb)


def _mlp_body(h_ref, p_ref, w1_ref, b1_ref, g1_ref, be1_ref,
              w2_ref, b2_ref, g2_ref, be2_ref, s_ref, out_ref):
    h = h_ref[...]
    bb = s_ref[0, 0] * h + p_ref[0, :N_NODES, :] + p_ref[1, :N_NODES, :]
    y = lax.dot_general(bb, w1_ref[...], (((1,), (1,)), ((), ())),
                        preferred_element_type=jnp.float32)
    y = y + b1_ref[...]
    m = jnp.mean(y, axis=0, keepdims=True)
    v = jnp.mean((y - m) ** 2, axis=0, keepdims=True)
    y = (y - m) / jnp.sqrt(v + 1e-5) * g1_ref[...] + be1_ref[...]
    y = jnp.maximum(y, 0.0)
    z = lax.dot_general(y, w2_ref[...], (((1,), (1,)), ((), ())),
                        preferred_element_type=jnp.float32)
    z = z + b2_ref[...]
    m2 = jnp.mean(z, axis=0, keepdims=True)
    v2 = jnp.mean((z - m2) ** 2, axis=0, keepdims=True)
    z = (z - m2) / jnp.sqrt(v2 + 1e-5) * g2_ref[...] + be2_ref[...]
    z = jnp.where(s_ref[0, 1] != 0.0, jnp.maximum(z, 0.0), z)
    out_ref[...] = z


def _mlp(h, partials, W1, b1, g1, be1, W2, b2, g2, be2, scal):
    return pl.pallas_call(
        _mlp_body,
        out_shape=jax.ShapeDtypeStruct((N_NODES, EMB), jnp.float32),
    )(h, partials, W1, b1.reshape(1, -1), g1.reshape(1, -1),
      be1.reshape(1, -1), W2, b2.reshape(1, -1), g2.reshape(1, -1),
      be2.reshape(1, -1), scal)


def kernel(h, edge_index, edge_attr, bond_emb, W1, b1, g1, be1,
           W2, b2, g2, be2, eps_param, add_activation=True):
    src = edge_index[0].astype(jnp.int32)
    dst = edge_index[1].astype(jnp.int32)
    ea = edge_attr.astype(jnp.int32)
    cidx = ea[:, 0] * 64 + ea[:, 1] * 8 + ea[:, 2]

    ctable = _build_ctable(bond_emb)
    zeros = jnp.zeros((NPAD, EMB), jnp.float32)
    partials = _sc_aggregate(h, src, cidx, dst, ctable, zeros)

    scal = jnp.stack([1.0 + eps_param,
                      jnp.asarray(add_activation, jnp.float32)]).reshape(1, 2)
    return _mlp(h, partials, W1, b1, g1, be1, W2, b2, g2, be2, scal)
